# Initial kernel scaffold; baseline (speedup 1.0000x reference)
import jax
import jax.numpy as jnp
from jax.experimental import pallas as pl


def kernel(graph_feats, edge_index, node2graph, W1, b1, R1, rb1, W2, b2, R2, rb2, W3, b3, R3, rb3, Wm1, bm1, Wm2, bm2):
    raise NotImplementedError("write your pallas kernel here")


# speedup vs baseline: 3.1099x; 3.1099x over previous
import functools

import jax
import jax.numpy as jnp
from jax import lax
from jax.experimental import pallas as pl
from jax.experimental.pallas import tpu as pltpu
from jax.experimental.pallas import tpu_sc as plsc

_N, _E, _D, _G = 10000, 320000, 128, 256
_MLP_H = 128
_NPAD = 10240
_NCHUNK = 80
_CW = 128
_EPAD = 32 * _NCHUNK * _CW
_GPAD = 264
_ROWS_PER_TILE = _NPAD // 16

_MESH = plsc.VectorSubcoreMesh(core_axis_name="c", subcore_axis_name="s")


@functools.partial(
    pl.kernel,
    out_type=jax.ShapeDtypeStruct((2, _NPAD, _D), jnp.float32),
    mesh=_MESH,
    scratch_types=[
        pltpu.VMEM_SHARED((_NPAD, _D), jnp.float32),
        pltpu.VMEM((_NCHUNK, _CW), jnp.int32),
        pltpu.VMEM((_NCHUNK, _CW), jnp.int32),
        pltpu.VMEM((_CW, _D), jnp.float32),
    ],
)
def _sc_aggregate(h_hbm, src_hbm, dst_hbm, zrows_hbm, out_hbm,
                  agg_sh, src_v, dst_v, rows_v):
    c = lax.axis_index("c")
    s = lax.axis_index("s")
    wid = c * 16 + s
    pltpu.sync_copy(src_hbm.at[wid], src_v)
    pltpu.sync_copy(dst_hbm.at[wid], dst_v)
    pltpu.sync_copy(zrows_hbm, agg_sh.at[pl.ds(s * _ROWS_PER_TILE, _ROWS_PER_TILE)])
    plsc.subcore_barrier()

    def step(j, carry):
        pltpu.sync_copy(h_hbm.at[src_v.at[j]], rows_v)
        pltpu.sync_copy(rows_v, agg_sh.at[dst_v.at[j]], add=True)
        return carry

    lax.fori_loop(0, _NCHUNK, step, 0)
    plsc.subcore_barrier()
    pltpu.sync_copy(agg_sh.at[pl.ds(s * _ROWS_PER_TILE, _ROWS_PER_TILE)],
                    out_hbm.at[c, pl.ds(s * _ROWS_PER_TILE, _ROWS_PER_TILE)])


@functools.partial(
    pl.kernel,
    out_type=jax.ShapeDtypeStruct((2, _G, _D), jnp.float32),
    mesh=_MESH,
    scratch_types=[
        pltpu.VMEM_SHARED((_GPAD, _D), jnp.float32),
        pltpu.VMEM((4, 80), jnp.int32),
        pltpu.VMEM((80, _D), jnp.float32),
    ],
)
def _sc_pool(h_hbm, n2g_hbm, zpool_hbm, out_hbm, pool_sh, n2g_v, rows_v):
    c = lax.axis_index("c")
    s = lax.axis_index("s")
    wid = c * 16 + s
    pltpu.sync_copy(n2g_hbm.at[wid], n2g_v)

    @pl.when(s == 0)
    def _():
        pltpu.sync_copy(zpool_hbm, pool_sh)

    plsc.subcore_barrier()

    def step(k, carry):
        base = wid * 320 + k * 80
        pltpu.sync_copy(h_hbm.at[pl.ds(base, 80)], rows_v)
        pltpu.sync_copy(rows_v, pool_sh.at[n2g_v.at[k]], add=True)
        return carry

    lax.fori_loop(0, 4, step, 0)
    plsc.subcore_barrier()

    @pl.when(s == 0)
    def _():
        pltpu.sync_copy(pool_sh.at[pl.ds(0, _G)], out_hbm.at[c])


_BR = 1024


def _tc_layer_body(aggs_ref, h_ref, w_ref, b_ref, r_ref, rb_ref, o_ref):
    a = aggs_ref[0] + aggs_ref[1]
    conv = jnp.dot(a, w_ref[...], preferred_element_type=jnp.float32) + b_ref[...]
    res = jnp.dot(h_ref[...], r_ref[...], preferred_element_type=jnp.float32) + rb_ref[...]
    o_ref[...] = jnp.maximum(conv, 0.0) + jnp.maximum(res, 0.0)


_tc_layer = pl.pallas_call(
    _tc_layer_body,
    grid=(_NPAD // _BR,),
    in_specs=[
        pl.BlockSpec((2, _BR, _D), lambda i: (0, i, 0)),
        pl.BlockSpec((_BR, _D), lambda i: (i, 0)),
        pl.BlockSpec((_D, _D), lambda i: (0, 0)),
        pl.BlockSpec((1, _D), lambda i: (0, 0)),
        pl.BlockSpec((_D, _D), lambda i: (0, 0)),
        pl.BlockSpec((1, _D), lambda i: (0, 0)),
    ],
    out_specs=pl.BlockSpec((_BR, _D), lambda i: (i, 0)),
    out_shape=jax.ShapeDtypeStruct((_NPAD, _D), jnp.float32),
)


def _tc_mlp_body(pools_ref, wm1_ref, bm1_ref, wm2_ref, bm2_ref, o_ref):
    p = pools_ref[0] + pools_ref[1]
    mid = jnp.maximum(
        jnp.dot(p, wm1_ref[...], preferred_element_type=jnp.float32) + bm1_ref[...],
        0.0)
    o_ref[...] = jnp.dot(mid, wm2_ref[...],
                         preferred_element_type=jnp.float32) + bm2_ref[...]


_tc_mlp = pl.pallas_call(
    _tc_mlp_body,
    out_shape=jax.ShapeDtypeStruct((_G, 1), jnp.float32),
)


def kernel(graph_feats, edge_index, node2graph,
           W1, b1, R1, rb1, W2, b2, R2, rb2, W3, b3, R3, rb3,
           Wm1, bm1, Wm2, bm2):
    f32 = jnp.float32
    h = jnp.concatenate(
        [graph_feats, jnp.zeros((_NPAD - _N, _D), f32)], axis=0)
    epad = jnp.full((_EPAD - _E,), _N, jnp.int32)
    srcr = jnp.concatenate([edge_index[0], epad]).reshape(32, _NCHUNK, _CW)
    dstr = jnp.concatenate([edge_index[1], epad]).reshape(32, _NCHUNK, _CW)
    n2gr = jnp.concatenate(
        [node2graph, jnp.full((_NPAD - _N,), _G, jnp.int32)]).reshape(32, 4, 80)
    zrows = jnp.zeros((_ROWS_PER_TILE, _D), f32)
    zpool = jnp.zeros((_GPAD, _D), f32)

    for (W, b, R, rb) in ((W1, b1, R1, rb1), (W2, b2, R2, rb2),
                          (W3, b3, R3, rb3)):
        aggs = _sc_aggregate(h, srcr, dstr, zrows)
        h = _tc_layer(aggs, h, W, b.reshape(1, _D), R, rb.reshape(1, _D))
    pools = _sc_pool(h, n2gr, zpool)
    return _tc_mlp(pools, Wm1, bm1.reshape(1, _MLP_H), Wm2, bm2.reshape(1, 1))
